# baseline (speedup 1.0000x reference)
import functools

import jax
import jax.numpy as jnp
from jax import lax
from jax.experimental import pallas as pl
from jax.experimental.pallas import tpu as pltpu
from jax.experimental.pallas import tpu_sc as plsc

D = 256
DM = 128
BLK = 1024
EPS = 1e-5


def _layer_norm(x, g, b):
    mu = jnp.mean(x, axis=-1, keepdims=True)
    xc = x - mu
    var = jnp.mean(xc * xc, axis=-1, keepdims=True)
    return xc * jax.lax.rsqrt(var + EPS) * g + b


def _dense_body(msgs_ref, hidden_ref, Wip_ref, bip_ref, Wv_ref, bv_ref,
                Wout_ref, bout_ref, g1_ref, beta1_ref, g2_ref, beta2_ref,
                Wf1_ref, bf1_ref, Wf2_ref, bf2_ref, out_ref, wc_ref, bc_ref):
    @pl.when(pl.program_id(0) == 0)
    def _combine():
        t = jax.lax.dot_general(Wv_ref[...], Wip_ref[...], (((1,), (0,)), ((), ())),
                                preferred_element_type=jnp.float32)
        wc_ref[...] = jax.lax.dot_general(Wout_ref[...], t, (((1,), (0,)), ((), ())),
                                          preferred_element_type=jnp.float32)
        tb = jax.lax.dot_general(bip_ref[...], Wv_ref[...], (((1,), (1,)), ((), ())),
                                 preferred_element_type=jnp.float32) + bv_ref[...]
        bc_ref[...] = jax.lax.dot_general(tb, Wout_ref[...], (((1,), (1,)), ((), ())),
                                          preferred_element_type=jnp.float32) + bout_ref[...]

    attn = jax.lax.dot_general(msgs_ref[...], wc_ref[...], (((1,), (1,)), ((), ())),
                               preferred_element_type=jnp.float32) + bc_ref[...]
    h1 = _layer_norm(hidden_ref[...] + attn, g1_ref[...], beta1_ref[...])
    f = jax.lax.dot_general(h1, Wf1_ref[...], (((1,), (1,)), ((), ())),
                            preferred_element_type=jnp.float32) + bf1_ref[...]
    f = jnp.maximum(f, 0.0)
    ffn = jax.lax.dot_general(f, Wf2_ref[...], (((1,), (1,)), ((), ())),
                              preferred_element_type=jnp.float32) + bf2_ref[...]
    out_ref[...] = _layer_norm(h1 + ffn, g2_ref[...], beta2_ref[...])


def _dense_update(msgs, hidden, W_ip, b_ip, W_v, b_v, W_out, b_out,
                  g1, beta1, g2, beta2, Wf1, bf1, Wf2, bf2):
    B = msgs.shape[0]
    grid = B // BLK
    row = lambda a: a.reshape(1, -1)
    full = lambda a: pl.BlockSpec(a.shape, lambda i: (0, 0))
    return pl.pallas_call(
        _dense_body,
        grid=(grid,),
        in_specs=[
            pl.BlockSpec((BLK, DM), lambda i: (i, 0)),
            pl.BlockSpec((BLK, D), lambda i: (i, 0)),
            full(W_ip), full(row(b_ip)), full(W_v), full(row(b_v)),
            full(W_out), full(row(b_out)), full(row(g1)), full(row(beta1)),
            full(row(g2)), full(row(beta2)), full(Wf1), full(row(bf1)),
            full(Wf2), full(row(bf2)),
        ],
        out_specs=pl.BlockSpec((BLK, D), lambda i: (i, 0)),
        out_shape=jax.ShapeDtypeStruct((B, D), jnp.float32),
        scratch_shapes=[pltpu.VMEM((D, DM), jnp.float32),
                        pltpu.VMEM((1, D), jnp.float32)],
    )(msgs, hidden, W_ip, row(b_ip), W_v, row(b_v), W_out, row(b_out),
      row(g1), row(beta1), row(g2), row(beta2), Wf1, row(bf1), Wf2, row(bf2))


BLKM = 10000


def _lower_bound(ids_ref, n, v):
    pos = jnp.int32(0)
    length = n
    while length > 1:
        half = length // 2
        pos = jnp.where(ids_ref[pos + half - 1] < v, pos + half, pos)
        length -= half
    return pos + jnp.where(ids_ref[pos] < v, 1, 0)


def _scatter_body(ids_ref, in_ref, nm_ref, out_ref):
    i = pl.program_id(0)
    out_ref[...] = in_ref[...]
    base = i * BLKM
    n = nm_ref.shape[0]
    s = _lower_bound(ids_ref, n, base)
    e = _lower_bound(ids_ref, n, base + BLKM)

    def _row(j, _):
        rel = ids_ref[j] - base
        out_ref[pl.ds(rel, 1), :] = nm_ref[pl.ds(j, 1), :]
        return 0

    lax.fori_loop(s, e, _row, 0)


def _copy_scatter(memory, ids, new_mem):
    M = memory.shape[0]
    grid = M // BLKM
    return pl.pallas_call(
        _scatter_body,
        grid_spec=pltpu.PrefetchScalarGridSpec(
            num_scalar_prefetch=1,
            grid=(grid,),
            in_specs=[
                pl.BlockSpec((BLKM, D), lambda i, *_: (i, 0)),
                pl.BlockSpec(new_mem.shape, lambda i, *_: (0, 0)),
            ],
            out_specs=pl.BlockSpec((BLKM, D), lambda i, *_: (i, 0)),
        ),
        out_shape=jax.ShapeDtypeStruct(memory.shape, memory.dtype),
    )(ids, memory, new_mem)


_NC, _NS, _L = 2, 16, 16
_NW = _NC * _NS
_B = 4096
_BPW = _B // _NW
_LU_PAD = 100096
_LPW = _LU_PAD // _NW


def _sc_body(mem_hbm, ids_hbm, ts_hbm, lu_hbm, hid_out, lu_out,
             idx_v, rows_v, ids_all, ts_all, lubuf, sem):
    wid = lax.axis_index("s") * _NC + lax.axis_index("c")
    base = wid * _BPW
    pltpu.sync_copy(ids_hbm.at[pl.ds(base, _BPW)], idx_v)
    gather = pltpu.async_copy(mem_hbm.at[idx_v], rows_v, sem)
    lbase = wid * _LPW
    pltpu.sync_copy(lu_hbm.at[pl.ds(lbase, _LPW)], lubuf)
    pltpu.sync_copy(ids_hbm, ids_all.at[pl.ds(0, _B)])
    ids_all[pl.ds(_B, _L)] = jnp.full((_L,), -1, jnp.int32)
    pltpu.sync_copy(ts_hbm, ts_all)

    def _probe(p):
        return ids_all[pl.ds(p, _L)][0]

    def _lb(v):
        pos = jnp.int32(0)
        length = _B
        while length > 1:
            half = length // 2
            pos = jnp.where(_probe(pos + half - 1) < v, pos + half, pos)
            length -= half
        return pos + jnp.where(_probe(pos) < v, 1, 0)

    s = _lb(lbase)
    e = _lb(lbase + _LPW)

    def _chunk(c, _):
        o = c * _L
        idv = ids_all[pl.ds(o, _L)]
        idnxt = ids_all[pl.ds(o + 1, _L)]
        tsv = ts_all[pl.ds(o, _L)]
        m = (idv >= lbase) & (idv < lbase + _LPW) & (idv != idnxt)
        plsc.store_scatter(lubuf, [idv - lbase], tsv, mask=m)
        return 0

    lax.fori_loop(s // _L, (e + _L - 1) // _L, _chunk, 0)
    pltpu.sync_copy(lubuf, lu_out.at[pl.ds(lbase, _LPW)])
    gather.wait()
    pltpu.sync_copy(rows_v, hid_out.at[pl.ds(base, _BPW)])


def _sc_gather_lu(memory, ids, timestamps, lu_padded):
    mesh = plsc.VectorSubcoreMesh(core_axis_name="c", subcore_axis_name="s")
    return pl.kernel(
        _sc_body,
        out_type=(jax.ShapeDtypeStruct((_B, D), jnp.float32),
                  jax.ShapeDtypeStruct((_LU_PAD,), jnp.int32)),
        mesh=mesh,
        scratch_types=[
            pltpu.VMEM((_BPW,), jnp.int32),
            pltpu.VMEM((_BPW, D), jnp.float32),
            pltpu.VMEM((_B + _L,), jnp.int32),
            pltpu.VMEM((_B,), jnp.int32),
            pltpu.VMEM((_LPW,), jnp.int32),
            pltpu.SemaphoreType.DMA,
        ],
        compiler_params=pltpu.CompilerParams(needs_layout_passes=False),
    )(memory, ids, timestamps, lu_padded)


def kernel(memory, unique_messages, unique_node_ids, last_update, timestamps,
           W_ip, b_ip, W_in, b_in, W_out, b_out, g1, beta1, g2, beta2,
           Wf1, bf1, Wf2, bf2):
    W_v = W_in[2 * D:]
    b_v = b_in[2 * D:]
    ids = unique_node_ids
    lu_padded = jnp.pad(last_update, (0, _LU_PAD - last_update.shape[0]))
    hidden, lu_out = _sc_gather_lu(memory, ids, timestamps, lu_padded)
    new_mem = _dense_update(unique_messages, hidden, W_ip, b_ip, W_v, b_v,
                            W_out, b_out, g1, beta1, g2, beta2, Wf1, bf1, Wf2, bf2)
    updated_memory = _copy_scatter(memory, ids, new_mem)
    return (updated_memory, lu_out[:last_update.shape[0]])


# speedup vs baseline: 1.0018x; 1.0018x over previous
import functools

import jax
import jax.numpy as jnp
from jax import lax
from jax.experimental import pallas as pl
from jax.experimental.pallas import tpu as pltpu
from jax.experimental.pallas import tpu_sc as plsc

D = 256
DM = 128
BLK = 1024
EPS = 1e-5


def _layer_norm(x, g, b):
    mu = jnp.mean(x, axis=-1, keepdims=True)
    xc = x - mu
    var = jnp.mean(xc * xc, axis=-1, keepdims=True)
    return xc * jax.lax.rsqrt(var + EPS) * g + b


def _dense_body(msgs_ref, hidden_ref, Wip_ref, bip_ref, Wv_ref, bv_ref,
                Wout_ref, bout_ref, g1_ref, beta1_ref, g2_ref, beta2_ref,
                Wf1_ref, bf1_ref, Wf2_ref, bf2_ref, out_ref, wc_ref, bc_ref):
    @pl.when(pl.program_id(0) == 0)
    def _combine():
        t = jax.lax.dot_general(Wv_ref[...], Wip_ref[...], (((1,), (0,)), ((), ())),
                                preferred_element_type=jnp.float32)
        wc_ref[...] = jax.lax.dot_general(Wout_ref[...], t, (((1,), (0,)), ((), ())),
                                          preferred_element_type=jnp.float32)
        tb = jax.lax.dot_general(bip_ref[...], Wv_ref[...], (((1,), (1,)), ((), ())),
                                 preferred_element_type=jnp.float32) + bv_ref[...]
        bc_ref[...] = jax.lax.dot_general(tb, Wout_ref[...], (((1,), (1,)), ((), ())),
                                          preferred_element_type=jnp.float32) + bout_ref[...]

    attn = jax.lax.dot_general(msgs_ref[...], wc_ref[...], (((1,), (1,)), ((), ())),
                               preferred_element_type=jnp.float32) + bc_ref[...]
    h1 = _layer_norm(hidden_ref[...] + attn, g1_ref[...], beta1_ref[...])
    bf16 = jnp.bfloat16
    f = jax.lax.dot_general(h1.astype(bf16), Wf1_ref[...].astype(bf16),
                            (((1,), (1,)), ((), ())),
                            preferred_element_type=jnp.float32) + bf1_ref[...]
    f = jnp.maximum(f, 0.0)
    ffn = jax.lax.dot_general(f.astype(bf16), Wf2_ref[...].astype(bf16),
                              (((1,), (1,)), ((), ())),
                              preferred_element_type=jnp.float32) + bf2_ref[...]
    out_ref[...] = _layer_norm(h1 + ffn, g2_ref[...], beta2_ref[...])


def _dense_update(msgs, hidden, W_ip, b_ip, W_v, b_v, W_out, b_out,
                  g1, beta1, g2, beta2, Wf1, bf1, Wf2, bf2):
    B = msgs.shape[0]
    grid = B // BLK
    row = lambda a: a.reshape(1, -1)
    full = lambda a: pl.BlockSpec(a.shape, lambda i: (0, 0))
    return pl.pallas_call(
        _dense_body,
        grid=(grid,),
        in_specs=[
            pl.BlockSpec((BLK, DM), lambda i: (i, 0)),
            pl.BlockSpec((BLK, D), lambda i: (i, 0)),
            full(W_ip), full(row(b_ip)), full(W_v), full(row(b_v)),
            full(W_out), full(row(b_out)), full(row(g1)), full(row(beta1)),
            full(row(g2)), full(row(beta2)), full(Wf1), full(row(bf1)),
            full(Wf2), full(row(bf2)),
        ],
        out_specs=pl.BlockSpec((BLK, D), lambda i: (i, 0)),
        out_shape=jax.ShapeDtypeStruct((B, D), jnp.float32),
        scratch_shapes=[pltpu.VMEM((D, DM), jnp.float32),
                        pltpu.VMEM((1, D), jnp.float32)],
    )(msgs, hidden, W_ip, row(b_ip), W_v, row(b_v), W_out, row(b_out),
      row(g1), row(beta1), row(g2), row(beta2), Wf1, row(bf1), Wf2, row(bf2))


BLKM = 10000


def _lower_bound(ids_ref, n, v):
    pos = jnp.int32(0)
    length = n
    while length > 1:
        half = length // 2
        pos = jnp.where(ids_ref[pos + half - 1] < v, pos + half, pos)
        length -= half
    return pos + jnp.where(ids_ref[pos] < v, 1, 0)


def _scatter_body(ids_ref, in_ref, nm_ref, out_ref):
    i = pl.program_id(0)
    out_ref[...] = in_ref[...]
    base = i * BLKM
    n = nm_ref.shape[0]
    s = _lower_bound(ids_ref, n, base)
    e = _lower_bound(ids_ref, n, base + BLKM)

    def _row(j, _):
        rel = ids_ref[j] - base
        out_ref[pl.ds(rel, 1), :] = nm_ref[pl.ds(j, 1), :]
        return 0

    lax.fori_loop(s, e, _row, 0)


def _copy_scatter(memory, ids, new_mem):
    M = memory.shape[0]
    grid = M // BLKM
    return pl.pallas_call(
        _scatter_body,
        grid_spec=pltpu.PrefetchScalarGridSpec(
            num_scalar_prefetch=1,
            grid=(grid,),
            in_specs=[
                pl.BlockSpec((BLKM, D), lambda i, *_: (i, 0)),
                pl.BlockSpec(new_mem.shape, lambda i, *_: (0, 0)),
            ],
            out_specs=pl.BlockSpec((BLKM, D), lambda i, *_: (i, 0)),
        ),
        out_shape=jax.ShapeDtypeStruct(memory.shape, memory.dtype),
    )(ids, memory, new_mem)


_NC, _NS, _L = 2, 16, 16
_NW = _NC * _NS
_B = 4096
_BPW = _B // _NW
_LU_PAD = 100096
_LPW = _LU_PAD // _NW


def _sc_body(mem_hbm, ids_hbm, ts_hbm, lu_hbm, hid_out, lu_out,
             idx_v, rows_v, ids_all, ts_all, lubuf, sem):
    wid = lax.axis_index("s") * _NC + lax.axis_index("c")
    base = wid * _BPW
    pltpu.sync_copy(ids_hbm.at[pl.ds(base, _BPW)], idx_v)
    gather = pltpu.async_copy(mem_hbm.at[idx_v], rows_v, sem)
    lbase = wid * _LPW
    pltpu.sync_copy(lu_hbm.at[pl.ds(lbase, _LPW)], lubuf)
    pltpu.sync_copy(ids_hbm, ids_all.at[pl.ds(0, _B)])
    ids_all[pl.ds(_B, _L)] = jnp.full((_L,), -1, jnp.int32)
    pltpu.sync_copy(ts_hbm, ts_all)

    def _probe(p):
        return ids_all[pl.ds(p, _L)][0]

    def _lb(v):
        pos = jnp.int32(0)
        length = _B
        while length > 1:
            half = length // 2
            pos = jnp.where(_probe(pos + half - 1) < v, pos + half, pos)
            length -= half
        return pos + jnp.where(_probe(pos) < v, 1, 0)

    s = _lb(lbase)
    e = _lb(lbase + _LPW)

    def _chunk(c, _):
        o = c * _L
        idv = ids_all[pl.ds(o, _L)]
        idnxt = ids_all[pl.ds(o + 1, _L)]
        tsv = ts_all[pl.ds(o, _L)]
        m = (idv >= lbase) & (idv < lbase + _LPW) & (idv != idnxt)
        plsc.store_scatter(lubuf, [idv - lbase], tsv, mask=m)
        return 0

    lax.fori_loop(s // _L, (e + _L - 1) // _L, _chunk, 0)
    pltpu.sync_copy(lubuf, lu_out.at[pl.ds(lbase, _LPW)])
    gather.wait()
    pltpu.sync_copy(rows_v, hid_out.at[pl.ds(base, _BPW)])


def _sc_gather_lu(memory, ids, timestamps, lu_padded):
    mesh = plsc.VectorSubcoreMesh(core_axis_name="c", subcore_axis_name="s")
    return pl.kernel(
        _sc_body,
        out_type=(jax.ShapeDtypeStruct((_B, D), jnp.float32),
                  jax.ShapeDtypeStruct((_LU_PAD,), jnp.int32)),
        mesh=mesh,
        scratch_types=[
            pltpu.VMEM((_BPW,), jnp.int32),
            pltpu.VMEM((_BPW, D), jnp.float32),
            pltpu.VMEM((_B + _L,), jnp.int32),
            pltpu.VMEM((_B,), jnp.int32),
            pltpu.VMEM((_LPW,), jnp.int32),
            pltpu.SemaphoreType.DMA,
        ],
        compiler_params=pltpu.CompilerParams(needs_layout_passes=False),
    )(memory, ids, timestamps, lu_padded)


def kernel(memory, unique_messages, unique_node_ids, last_update, timestamps,
           W_ip, b_ip, W_in, b_in, W_out, b_out, g1, beta1, g2, beta2,
           Wf1, bf1, Wf2, bf2):
    W_v = W_in[2 * D:]
    b_v = b_in[2 * D:]
    ids = unique_node_ids
    lu_padded = jnp.pad(last_update, (0, _LU_PAD - last_update.shape[0]))
    hidden, lu_out = _sc_gather_lu(memory, ids, timestamps, lu_padded)
    new_mem = _dense_update(unique_messages, hidden, W_ip, b_ip, W_v, b_v,
                            W_out, b_out, g1, beta1, g2, beta2, Wf1, bf1, Wf2, bf2)
    updated_memory = _copy_scatter(memory, ids, new_mem)
    return (updated_memory, lu_out[:last_update.shape[0]])


# speedup vs baseline: 1.0344x; 1.0325x over previous
import functools

import jax
import jax.numpy as jnp
from jax import lax
from jax.experimental import pallas as pl
from jax.experimental.pallas import tpu as pltpu
from jax.experimental.pallas import tpu_sc as plsc

D = 256
DM = 128
BLK = 1024
EPS = 1e-5


def _layer_norm(x, g, b):
    mu = jnp.mean(x, axis=-1, keepdims=True)
    xc = x - mu
    var = jnp.mean(xc * xc, axis=-1, keepdims=True)
    return xc * jax.lax.rsqrt(var + EPS) * g + b


def _dense_body(msgs_ref, hidden_ref, Wip_ref, bip_ref, Wv_ref, bv_ref,
                Wout_ref, bout_ref, g1_ref, beta1_ref, g2_ref, beta2_ref,
                Wf1_ref, bf1_ref, Wf2_ref, bf2_ref, out_ref, wc_ref, bc_ref):
    @pl.when(pl.program_id(0) == 0)
    def _combine():
        t = jax.lax.dot_general(Wv_ref[...], Wip_ref[...], (((1,), (0,)), ((), ())),
                                preferred_element_type=jnp.float32)
        wc_ref[...] = jax.lax.dot_general(Wout_ref[...], t, (((1,), (0,)), ((), ())),
                                          preferred_element_type=jnp.float32)
        tb = jax.lax.dot_general(bip_ref[...], Wv_ref[...], (((1,), (1,)), ((), ())),
                                 preferred_element_type=jnp.float32) + bv_ref[...]
        bc_ref[...] = jax.lax.dot_general(tb, Wout_ref[...], (((1,), (1,)), ((), ())),
                                          preferred_element_type=jnp.float32) + bout_ref[...]

    attn = jax.lax.dot_general(msgs_ref[...], wc_ref[...], (((1,), (1,)), ((), ())),
                               preferred_element_type=jnp.float32) + bc_ref[...]
    h1 = _layer_norm(hidden_ref[...] + attn, g1_ref[...], beta1_ref[...])
    f = jax.lax.dot_general(h1, Wf1_ref[...], (((1,), (1,)), ((), ())),
                            preferred_element_type=jnp.float32) + bf1_ref[...]
    f = jnp.maximum(f, 0.0)
    ffn = jax.lax.dot_general(f, Wf2_ref[...], (((1,), (1,)), ((), ())),
                              preferred_element_type=jnp.float32) + bf2_ref[...]
    out_ref[...] = _layer_norm(h1 + ffn, g2_ref[...], beta2_ref[...])


def _dense_update(msgs, hidden, W_ip, b_ip, W_v, b_v, W_out, b_out,
                  g1, beta1, g2, beta2, Wf1, bf1, Wf2, bf2):
    B = msgs.shape[0]
    grid = B // BLK
    row = lambda a: a.reshape(1, -1)
    full = lambda a: pl.BlockSpec(a.shape, lambda i: (0, 0))
    return pl.pallas_call(
        _dense_body,
        grid=(grid,),
        in_specs=[
            pl.BlockSpec((BLK, DM), lambda i: (i, 0)),
            pl.BlockSpec((BLK, D), lambda i: (i, 0)),
            full(W_ip), full(row(b_ip)), full(W_v), full(row(b_v)),
            full(W_out), full(row(b_out)), full(row(g1)), full(row(beta1)),
            full(row(g2)), full(row(beta2)), full(Wf1), full(row(bf1)),
            full(Wf2), full(row(bf2)),
        ],
        out_specs=pl.BlockSpec((BLK, D), lambda i: (i, 0)),
        out_shape=jax.ShapeDtypeStruct((B, D), jnp.float32),
        scratch_shapes=[pltpu.VMEM((D, DM), jnp.float32),
                        pltpu.VMEM((1, D), jnp.float32)],
    )(msgs, hidden, W_ip, row(b_ip), W_v, row(b_v), W_out, row(b_out),
      row(g1), row(beta1), row(g2), row(beta2), Wf1, row(bf1), Wf2, row(bf2))


BLKM = 10000


def _lower_bound(ids_ref, n, v):
    pos = jnp.int32(0)
    length = n
    while length > 1:
        half = length // 2
        pos = jnp.where(ids_ref[pos + half - 1] < v, pos + half, pos)
        length -= half
    return pos + jnp.where(ids_ref[pos] < v, 1, 0)


def _scatter_body(ids_ref, in_ref, nm_ref, out_ref):
    i = pl.program_id(0)
    out_ref[...] = in_ref[...]
    base = i * BLKM
    n = nm_ref.shape[0]
    s = _lower_bound(ids_ref, n, base)
    e = _lower_bound(ids_ref, n, base + BLKM)

    def _row(j, _):
        rel = ids_ref[j] - base
        out_ref[pl.ds(rel, 1), :] = nm_ref[pl.ds(j, 1), :]
        return 0

    lax.fori_loop(s, e, _row, 0)


def _copy_scatter(memory, ids, new_mem):
    M = memory.shape[0]
    grid = M // BLKM
    return pl.pallas_call(
        _scatter_body,
        grid_spec=pltpu.PrefetchScalarGridSpec(
            num_scalar_prefetch=1,
            grid=(grid,),
            in_specs=[
                pl.BlockSpec((BLKM, D), lambda i, *_: (i, 0)),
                pl.BlockSpec(new_mem.shape, lambda i, *_: (0, 0)),
            ],
            out_specs=pl.BlockSpec((BLKM, D), lambda i, *_: (i, 0)),
        ),
        out_shape=jax.ShapeDtypeStruct(memory.shape, memory.dtype),
    )(ids, memory, new_mem)


_NC, _NS, _L = 2, 16, 16
_NW = _NC * _NS
_B = 4096
_BPW = _B // _NW
_LU_PAD = 100096
_LPW = _LU_PAD // _NW


def _sc_gather_body(mem_hbm, ids_hbm, hid_out, idx_v, rows_v, sem):
    wid = lax.axis_index("s") * _NC + lax.axis_index("c")
    base = wid * _BPW
    pltpu.sync_copy(ids_hbm.at[pl.ds(base, _BPW)], idx_v)
    pltpu.async_copy(mem_hbm.at[idx_v], rows_v, sem).wait()
    pltpu.sync_copy(rows_v, hid_out.at[pl.ds(base, _BPW)])


def _sc_gather(memory, ids):
    mesh = plsc.VectorSubcoreMesh(core_axis_name="c", subcore_axis_name="s")
    return pl.kernel(
        _sc_gather_body,
        out_type=jax.ShapeDtypeStruct((_B, D), jnp.float32),
        mesh=mesh,
        scratch_types=[
            pltpu.VMEM((_BPW,), jnp.int32),
            pltpu.VMEM((_BPW, D), jnp.float32),
            pltpu.SemaphoreType.DMA,
        ],
        compiler_params=pltpu.CompilerParams(needs_layout_passes=False),
    )(memory, ids)


def _sc_body(ids_hbm, ts_hbm, lu_hbm, lu_out,
             ids_all, ts_all, lubuf, sem):
    wid = lax.axis_index("s") * _NC + lax.axis_index("c")
    lbase = wid * _LPW
    pltpu.sync_copy(lu_hbm.at[pl.ds(lbase, _LPW)], lubuf)
    pltpu.sync_copy(ids_hbm, ids_all.at[pl.ds(0, _B)])
    ids_all[pl.ds(_B, _L)] = jnp.full((_L,), -1, jnp.int32)
    pltpu.sync_copy(ts_hbm, ts_all)

    def _probe(p):
        return ids_all[pl.ds(p, _L)][0]

    def _lb(v):
        pos = jnp.int32(0)
        length = _B
        while length > 1:
            half = length // 2
            pos = jnp.where(_probe(pos + half - 1) < v, pos + half, pos)
            length -= half
        return pos + jnp.where(_probe(pos) < v, 1, 0)

    s = _lb(lbase)
    e = _lb(lbase + _LPW)

    def _chunk(c, _):
        o = c * _L
        idv = ids_all[pl.ds(o, _L)]
        idnxt = ids_all[pl.ds(o + 1, _L)]
        tsv = ts_all[pl.ds(o, _L)]
        m = (idv >= lbase) & (idv < lbase + _LPW) & (idv != idnxt)
        plsc.store_scatter(lubuf, [idv - lbase], tsv, mask=m)
        return 0

    lax.fori_loop(s // _L, (e + _L - 1) // _L, _chunk, 0)
    pltpu.sync_copy(lubuf, lu_out.at[pl.ds(lbase, _LPW)])


def _sc_lu(ids, timestamps, lu_padded):
    mesh = plsc.VectorSubcoreMesh(core_axis_name="c", subcore_axis_name="s")
    return pl.kernel(
        _sc_body,
        out_type=jax.ShapeDtypeStruct((_LU_PAD,), jnp.int32),
        mesh=mesh,
        scratch_types=[
            pltpu.VMEM((_B + _L,), jnp.int32),
            pltpu.VMEM((_B,), jnp.int32),
            pltpu.VMEM((_LPW,), jnp.int32),
            pltpu.SemaphoreType.DMA,
        ],
        compiler_params=pltpu.CompilerParams(needs_layout_passes=False),
    )(ids, timestamps, lu_padded)


def kernel(memory, unique_messages, unique_node_ids, last_update, timestamps,
           W_ip, b_ip, W_in, b_in, W_out, b_out, g1, beta1, g2, beta2,
           Wf1, bf1, Wf2, bf2):
    W_v = W_in[2 * D:]
    b_v = b_in[2 * D:]
    ids = unique_node_ids
    lu_padded = jnp.pad(last_update, (0, _LU_PAD - last_update.shape[0]))
    hidden = _sc_gather(memory, ids)
    lu_out = _sc_lu(ids, timestamps, lu_padded)
    new_mem = _dense_update(unique_messages, hidden, W_ip, b_ip, W_v, b_v,
                            W_out, b_out, g1, beta1, g2, beta2, Wf1, bf1, Wf2, bf2)
    updated_memory = _copy_scatter(memory, ids, new_mem)
    return (updated_memory, lu_out[:last_update.shape[0]])
